# baseline (speedup 1.0000x reference)
import functools

import jax
import jax.numpy as jnp
from jax import lax
from jax.experimental import pallas as pl
from jax.experimental.pallas import tpu as pltpu
from jax.experimental.pallas import tpu_sc as plsc

LAMBDA_L = 0.01
LAMBDA_S = 0.01
EPS = 1e-08

B, H, W = 8, 512, 512
NA = 64
N = B * H * W
NC, NS, L = 2, 16, 16
NW = NC * NS
PER_TILE = N // NW
CHUNK = 16384
UNROLL = 4
NCHUNK = PER_TILE // CHUNK
TILES_PER_BATCH = NW // B

_mesh = plsc.VectorSubcoreMesh(core_axis_name="c", subcore_axis_name="s")
_params = pltpu.CompilerParams(needs_layout_passes=False)


@functools.partial(
    pl.kernel,
    mesh=_mesh,
    compiler_params=_params,
    out_type=jax.ShapeDtypeStruct((NW * NA,), jnp.float32),
    scratch_types=[
        pltpu.VMEM((CHUNK,), jnp.float32),
        pltpu.VMEM((CHUNK,), jnp.float32),
        pltpu.VMEM((CHUNK,), jnp.int32),
        pltpu.VMEM((L * NA,), jnp.float32),
        pltpu.VMEM((NA,), jnp.float32),
    ],
)
def _phase1(l_hbm, s_hbm, a_hbm, part_hbm, lbuf, sbuf, abuf, accum, sums):
    wid = lax.axis_index("c") * NS + lax.axis_index("s")
    base = wid * PER_TILE
    zero = jnp.zeros((L,), jnp.float32)
    for k in range(NA):
        accum[pl.ds(k * L, L)] = zero
    lane_off = jnp.arange(L, dtype=jnp.int32) * NA

    def chunk_body(ci, _):
        off = base + ci * CHUNK
        pltpu.sync_copy(l_hbm.at[pl.ds(off, CHUNK)], lbuf)
        pltpu.sync_copy(s_hbm.at[pl.ds(off, CHUNK)], sbuf)
        pltpu.sync_copy(a_hbm.at[pl.ds(off, CHUNK)], abuf)

        def body(i, _):
            for j in range(UNROLL):
                p = pl.ds(i * (L * UNROLL) + j * L, L)
                score = (lbuf[p] + LAMBDA_L) * (sbuf[p] + LAMBDA_S)
                plsc.addupdate_scatter(accum, [lane_off + abuf[p]], score)
            return 0

        lax.fori_loop(0, CHUNK // (L * UNROLL), body, 0)
        return 0

    lax.fori_loop(0, NCHUNK, chunk_body, 0)

    for k in range(NA // L):
        t = accum[pl.ds(k * L, L)]
        for lane in range(1, L):
            t = t + accum[pl.ds(lane * NA + k * L, L)]
        sums[pl.ds(k * L, L)] = t
    pltpu.sync_copy(sums, part_hbm.at[pl.ds(wid * NA, NA)])


@functools.partial(
    pl.kernel,
    mesh=_mesh,
    compiler_params=_params,
    out_type=jax.ShapeDtypeStruct((N,), jnp.float32),
    scratch_types=[
        pltpu.VMEM((CHUNK,), jnp.float32),
        pltpu.VMEM((CHUNK,), jnp.float32),
        pltpu.VMEM((CHUNK,), jnp.int32),
        pltpu.VMEM((CHUNK,), jnp.float32),
        pltpu.VMEM((TILES_PER_BATCH * NA,), jnp.float32),
        pltpu.VMEM((NA,), jnp.float32),
        pltpu.VMEM((NA,), jnp.float32),
    ],
)
def _phase2(l_hbm, s_hbm, a_hbm, part_hbm, c_hbm, out_hbm,
            lbuf, sbuf, abuf, obuf, pbuf, cbuf, fbuf):
    wid = lax.axis_index("c") * NS + lax.axis_index("s")
    batch = wid // TILES_PER_BATCH
    pltpu.sync_copy(part_hbm.at[pl.ds(batch * TILES_PER_BATCH * NA,
                                      TILES_PER_BATCH * NA)], pbuf)
    pltpu.sync_copy(c_hbm, cbuf)
    for k in range(NA // L):
        t = pbuf[pl.ds(k * L, L)]
        for j in range(1, TILES_PER_BATCH):
            t = t + pbuf[pl.ds(j * NA + k * L, L)]
        fbuf[pl.ds(k * L, L)] = cbuf[pl.ds(k * L, L)] / (t + EPS)

    base = wid * PER_TILE

    def chunk_body(ci, _):
        off = base + ci * CHUNK
        pltpu.sync_copy(l_hbm.at[pl.ds(off, CHUNK)], lbuf)
        pltpu.sync_copy(s_hbm.at[pl.ds(off, CHUNK)], sbuf)
        pltpu.sync_copy(a_hbm.at[pl.ds(off, CHUNK)], abuf)

        def body(i, _):
            for j in range(UNROLL):
                p = pl.ds(i * (L * UNROLL) + j * L, L)
                score = (lbuf[p] + LAMBDA_L) * (sbuf[p] + LAMBDA_S)
                f = plsc.load_gather(fbuf, [abuf[p]])
                obuf[p] = score * f
            return 0

        lax.fori_loop(0, CHUNK // (L * UNROLL), body, 0)
        pltpu.sync_copy(obuf, out_hbm.at[pl.ds(off, CHUNK)])
        return 0

    lax.fori_loop(0, NCHUNK, chunk_body, 0)


def kernel(lights, settlement, admin_ids, census_totals):
    l_flat = lights.reshape(-1)
    s_flat = settlement.reshape(-1)
    a_flat = admin_ids.reshape(-1)
    partials = _phase1(l_flat, s_flat, a_flat)
    out = _phase2(l_flat, s_flat, a_flat, partials, census_totals)
    return out.reshape(lights.shape)


# speedup vs baseline: 1.0241x; 1.0241x over previous
import functools

import jax
import jax.numpy as jnp
from jax import lax
from jax.experimental import pallas as pl
from jax.experimental.pallas import tpu as pltpu
from jax.experimental.pallas import tpu_sc as plsc

LAMBDA_L = 0.01
LAMBDA_S = 0.01
EPS = 1e-08

B, H, W = 8, 512, 512
NA = 64
N = B * H * W
NC, NS, L = 2, 16, 16
NW = NC * NS
PER_TILE = N // NW
CHUNK = 16384
UNROLL = 4
NCHUNK = PER_TILE // CHUNK
TILES_PER_BATCH = NW // B

_mesh = plsc.VectorSubcoreMesh(core_axis_name="c", subcore_axis_name="s")
_params = pltpu.CompilerParams(needs_layout_passes=False)


@functools.partial(
    pl.kernel,
    mesh=_mesh,
    compiler_params=_params,
    out_type=jax.ShapeDtypeStruct((NW * NA,), jnp.float32),
    scratch_types=[
        pltpu.VMEM((CHUNK,), jnp.float32),
        pltpu.VMEM((CHUNK,), jnp.float32),
        pltpu.VMEM((CHUNK,), jnp.int32),
        pltpu.VMEM((L * NA,), jnp.float32),
        pltpu.VMEM((NA,), jnp.float32),
    ],
)
def _phase1(l_hbm, s_hbm, a_hbm, part_hbm, lbuf, sbuf, abuf, accum, sums):
    wid = lax.axis_index("c") * NS + lax.axis_index("s")
    base = wid * PER_TILE
    zero = jnp.zeros((L,), jnp.float32)
    for k in range(NA):
        accum[pl.ds(k * L, L)] = zero
    lane_iota = jnp.arange(L, dtype=jnp.int32)

    def chunk_body(ci, _):
        off = base + ci * CHUNK
        pltpu.sync_copy(l_hbm.at[pl.ds(off, CHUNK)], lbuf)
        pltpu.sync_copy(s_hbm.at[pl.ds(off, CHUNK)], sbuf)
        pltpu.sync_copy(a_hbm.at[pl.ds(off, CHUNK)], abuf)

        def body(i, _):
            for j in range(UNROLL):
                p = pl.ds(i * (L * UNROLL) + j * L, L)
                score = (lbuf[p] + LAMBDA_L) * (sbuf[p] + LAMBDA_S)
                plsc.addupdate_scatter(accum, [abuf[p] * L + lane_iota], score)
            return 0

        lax.fori_loop(0, CHUNK // (L * UNROLL), body, 0)
        return 0

    lax.fori_loop(0, NCHUNK, chunk_body, 0)

    stride_iota = lane_iota * L
    for k in range(NA // L):
        t = jnp.zeros((L,), jnp.float32)
        for lane in range(L):
            t = t + plsc.load_gather(accum, [stride_iota + (k * L * L + lane)])
        sums[pl.ds(k * L, L)] = t
    pltpu.sync_copy(sums, part_hbm.at[pl.ds(wid * NA, NA)])


@functools.partial(
    pl.kernel,
    mesh=_mesh,
    compiler_params=_params,
    out_type=jax.ShapeDtypeStruct((N,), jnp.float32),
    scratch_types=[
        pltpu.VMEM((CHUNK,), jnp.float32),
        pltpu.VMEM((CHUNK,), jnp.float32),
        pltpu.VMEM((CHUNK,), jnp.int32),
        pltpu.VMEM((CHUNK,), jnp.float32),
        pltpu.VMEM((TILES_PER_BATCH * NA,), jnp.float32),
        pltpu.VMEM((NA,), jnp.float32),
        pltpu.VMEM((NA,), jnp.float32),
    ],
)
def _phase2(l_hbm, s_hbm, a_hbm, part_hbm, c_hbm, out_hbm,
            lbuf, sbuf, abuf, obuf, pbuf, cbuf, fbuf):
    wid = lax.axis_index("c") * NS + lax.axis_index("s")
    batch = wid // TILES_PER_BATCH
    pltpu.sync_copy(part_hbm.at[pl.ds(batch * TILES_PER_BATCH * NA,
                                      TILES_PER_BATCH * NA)], pbuf)
    pltpu.sync_copy(c_hbm, cbuf)
    for k in range(NA // L):
        t = pbuf[pl.ds(k * L, L)]
        for j in range(1, TILES_PER_BATCH):
            t = t + pbuf[pl.ds(j * NA + k * L, L)]
        fbuf[pl.ds(k * L, L)] = cbuf[pl.ds(k * L, L)] / (t + EPS)

    base = wid * PER_TILE

    def chunk_body(ci, _):
        off = base + ci * CHUNK
        pltpu.sync_copy(l_hbm.at[pl.ds(off, CHUNK)], lbuf)
        pltpu.sync_copy(s_hbm.at[pl.ds(off, CHUNK)], sbuf)
        pltpu.sync_copy(a_hbm.at[pl.ds(off, CHUNK)], abuf)

        def body(i, _):
            for j in range(UNROLL):
                p = pl.ds(i * (L * UNROLL) + j * L, L)
                score = (lbuf[p] + LAMBDA_L) * (sbuf[p] + LAMBDA_S)
                f = plsc.load_gather(fbuf, [abuf[p]])
                obuf[p] = score * f
            return 0

        lax.fori_loop(0, CHUNK // (L * UNROLL), body, 0)
        pltpu.sync_copy(obuf, out_hbm.at[pl.ds(off, CHUNK)])
        return 0

    lax.fori_loop(0, NCHUNK, chunk_body, 0)


def kernel(lights, settlement, admin_ids, census_totals):
    l_flat = lights.reshape(-1)
    s_flat = settlement.reshape(-1)
    a_flat = admin_ids.reshape(-1)
    partials = _phase1(l_flat, s_flat, a_flat)
    out = _phase2(l_flat, s_flat, a_flat, partials, census_totals)
    return out.reshape(lights.shape)


# speedup vs baseline: 1.2239x; 1.1951x over previous
import functools

import jax
import jax.numpy as jnp
from jax import lax
from jax.experimental import pallas as pl
from jax.experimental.pallas import tpu as pltpu
from jax.experimental.pallas import tpu_sc as plsc

LAMBDA_L = 0.01
LAMBDA_S = 0.01
EPS = 1e-08

B, H, W = 8, 512, 512
NA = 64
N = B * H * W
NC, NS, L = 2, 16, 16
NW = NC * NS
PER_TILE = N // NW
UNROLL = 4
TILES_PER_BATCH = NW // B

CHUNK1 = 16384
NCHUNK1 = PER_TILE // CHUNK1
CHUNK2 = 8192
NCHUNK2 = PER_TILE // CHUNK2

_mesh = plsc.VectorSubcoreMesh(core_axis_name="c", subcore_axis_name="s")
_params = pltpu.CompilerParams(needs_layout_passes=False)


@functools.partial(
    pl.kernel,
    mesh=_mesh,
    compiler_params=_params,
    out_type=jax.ShapeDtypeStruct((NW * NA,), jnp.float32),
    scratch_types=[
        pltpu.VMEM((CHUNK1,), jnp.float32),
        pltpu.VMEM((CHUNK1,), jnp.float32),
        pltpu.VMEM((CHUNK1,), jnp.int32),
        pltpu.VMEM((CHUNK1,), jnp.float32),
        pltpu.VMEM((CHUNK1,), jnp.float32),
        pltpu.VMEM((CHUNK1,), jnp.int32),
        pltpu.VMEM((L * NA,), jnp.float32),
        pltpu.VMEM((NA,), jnp.float32),
        pltpu.SemaphoreType.DMA,
        pltpu.SemaphoreType.DMA,
    ],
)
def _phase1(l_hbm, s_hbm, a_hbm, part_hbm,
            lbuf0, sbuf0, abuf0, lbuf1, sbuf1, abuf1, accum, sums,
            sem0, sem1):
    wid = lax.axis_index("c") * NS + lax.axis_index("s")
    base = wid * PER_TILE
    lbufs, sbufs, abufs = (lbuf0, lbuf1), (sbuf0, sbuf1), (abuf0, abuf1)
    sems = (sem0, sem1)

    zero = jnp.zeros((L,), jnp.float32)
    for k in range(NA):
        accum[pl.ds(k * L, L)] = zero
    lane_iota = jnp.arange(L, dtype=jnp.int32)

    def issue(ci, slot):
        off = base + ci * CHUNK1
        return (
            pltpu.async_copy(l_hbm.at[pl.ds(off, CHUNK1)], lbufs[slot], sems[slot]),
            pltpu.async_copy(s_hbm.at[pl.ds(off, CHUNK1)], sbufs[slot], sems[slot]),
            pltpu.async_copy(a_hbm.at[pl.ds(off, CHUNK1)], abufs[slot], sems[slot]),
        )

    pending = {0: issue(0, 0)}
    for ci in range(NCHUNK1):
        slot = ci % 2
        if ci + 1 < NCHUNK1:
            pending[(ci + 1) % 2] = issue(ci + 1, (ci + 1) % 2)
        for h in pending[slot]:
            h.wait()
        lbuf, sbuf, abuf = lbufs[slot], sbufs[slot], abufs[slot]

        def body(i, _):
            for j in range(UNROLL):
                p = pl.ds(i * (L * UNROLL) + j * L, L)
                score = (lbuf[p] + LAMBDA_L) * (sbuf[p] + LAMBDA_S)
                plsc.addupdate_scatter(accum, [abuf[p] * L + lane_iota], score)
            return 0

        lax.fori_loop(0, CHUNK1 // (L * UNROLL), body, 0)

    stride_iota = lane_iota * L
    for k in range(NA // L):
        t = jnp.zeros((L,), jnp.float32)
        for lane in range(L):
            t = t + plsc.load_gather(accum, [stride_iota + (k * L * L + lane)])
        sums[pl.ds(k * L, L)] = t
    pltpu.sync_copy(sums, part_hbm.at[pl.ds(wid * NA, NA)])


@functools.partial(
    pl.kernel,
    mesh=_mesh,
    compiler_params=_params,
    out_type=jax.ShapeDtypeStruct((N,), jnp.float32),
    scratch_types=[
        pltpu.VMEM((CHUNK2,), jnp.float32),
        pltpu.VMEM((CHUNK2,), jnp.float32),
        pltpu.VMEM((CHUNK2,), jnp.int32),
        pltpu.VMEM((CHUNK2,), jnp.float32),
        pltpu.VMEM((CHUNK2,), jnp.float32),
        pltpu.VMEM((CHUNK2,), jnp.int32),
        pltpu.VMEM((CHUNK2,), jnp.float32),
        pltpu.VMEM((CHUNK2,), jnp.float32),
        pltpu.VMEM((TILES_PER_BATCH * NA,), jnp.float32),
        pltpu.VMEM((NA,), jnp.float32),
        pltpu.VMEM((NA,), jnp.float32),
        pltpu.SemaphoreType.DMA,
        pltpu.SemaphoreType.DMA,
        pltpu.SemaphoreType.DMA,
        pltpu.SemaphoreType.DMA,
    ],
)
def _phase2(l_hbm, s_hbm, a_hbm, part_hbm, c_hbm, out_hbm,
            lbuf0, sbuf0, abuf0, lbuf1, sbuf1, abuf1, obuf0, obuf1,
            pbuf, cbuf, fbuf, sem0, sem1, osem0, osem1):
    wid = lax.axis_index("c") * NS + lax.axis_index("s")
    batch = wid // TILES_PER_BATCH
    base = wid * PER_TILE
    lbufs, sbufs, abufs = (lbuf0, lbuf1), (sbuf0, sbuf1), (abuf0, abuf1)
    obufs, sems, osems = (obuf0, obuf1), (sem0, sem1), (osem0, osem1)

    def issue(ci, slot):
        off = base + ci * CHUNK2
        return (
            pltpu.async_copy(l_hbm.at[pl.ds(off, CHUNK2)], lbufs[slot], sems[slot]),
            pltpu.async_copy(s_hbm.at[pl.ds(off, CHUNK2)], sbufs[slot], sems[slot]),
            pltpu.async_copy(a_hbm.at[pl.ds(off, CHUNK2)], abufs[slot], sems[slot]),
        )

    pending = {0: issue(0, 0)}

    pltpu.sync_copy(part_hbm.at[pl.ds(batch * TILES_PER_BATCH * NA,
                                      TILES_PER_BATCH * NA)], pbuf)
    pltpu.sync_copy(c_hbm, cbuf)
    for k in range(NA // L):
        t = pbuf[pl.ds(k * L, L)]
        for j in range(1, TILES_PER_BATCH):
            t = t + pbuf[pl.ds(j * NA + k * L, L)]
        fbuf[pl.ds(k * L, L)] = cbuf[pl.ds(k * L, L)] / (t + EPS)

    out_pending = {0: None, 1: None}
    for ci in range(NCHUNK2):
        slot = ci % 2
        if ci + 1 < NCHUNK2:
            pending[(ci + 1) % 2] = issue(ci + 1, (ci + 1) % 2)
        for h in pending[slot]:
            h.wait()
        if out_pending[slot] is not None:
            out_pending[slot].wait()
        lbuf, sbuf, abuf, obuf = lbufs[slot], sbufs[slot], abufs[slot], obufs[slot]

        def body(i, _):
            for j in range(UNROLL):
                p = pl.ds(i * (L * UNROLL) + j * L, L)
                score = (lbuf[p] + LAMBDA_L) * (sbuf[p] + LAMBDA_S)
                f = plsc.load_gather(fbuf, [abuf[p]])
                obuf[p] = score * f
            return 0

        lax.fori_loop(0, CHUNK2 // (L * UNROLL), body, 0)
        off = base + ci * CHUNK2
        out_pending[slot] = pltpu.async_copy(
            obuf, out_hbm.at[pl.ds(off, CHUNK2)], osems[slot])
    for slot in (0, 1):
        if out_pending[slot] is not None:
            out_pending[slot].wait()


def kernel(lights, settlement, admin_ids, census_totals):
    l_flat = lights.reshape(-1)
    s_flat = settlement.reshape(-1)
    a_flat = admin_ids.reshape(-1)
    partials = _phase1(l_flat, s_flat, a_flat)
    out = _phase2(l_flat, s_flat, a_flat, partials, census_totals)
    return out.reshape(lights.shape)


# speedup vs baseline: 1.4873x; 1.2152x over previous
import functools

import jax
import jax.numpy as jnp
from jax import lax
from jax.experimental import pallas as pl
from jax.experimental.pallas import tpu as pltpu
from jax.experimental.pallas import tpu_sc as plsc

LAMBDA_L = 0.01
LAMBDA_S = 0.01
EPS = 1e-08

B, H, W = 8, 512, 512
NA = 64
NC, NS, L = 2, 16, 16
NW = NC * NS
ROWS_PER_TILE = B * H // NW
TILES_PER_BATCH = NW // B
UNROLL = 4

ROWS1 = 32
NBLK1 = ROWS_PER_TILE // ROWS1
ROWS2 = 16
NBLK2 = ROWS_PER_TILE // ROWS2
GPR = W // L

_mesh = plsc.VectorSubcoreMesh(core_axis_name="c", subcore_axis_name="s")
_params = pltpu.CompilerParams(needs_layout_passes=False)


@functools.partial(
    pl.kernel,
    mesh=_mesh,
    compiler_params=_params,
    out_type=jax.ShapeDtypeStruct((NW * NA,), jnp.float32),
    scratch_types=[
        pltpu.VMEM((ROWS1, W), jnp.float32),
        pltpu.VMEM((ROWS1, W), jnp.float32),
        pltpu.VMEM((ROWS1, W), jnp.int32),
        pltpu.VMEM((ROWS1, W), jnp.float32),
        pltpu.VMEM((ROWS1, W), jnp.float32),
        pltpu.VMEM((ROWS1, W), jnp.int32),
        pltpu.VMEM((L * NA,), jnp.float32),
        pltpu.VMEM((NA,), jnp.float32),
        pltpu.SemaphoreType.DMA,
        pltpu.SemaphoreType.DMA,
    ],
)
def _phase1(l_hbm, s_hbm, a_hbm, part_hbm,
            lbuf0, sbuf0, abuf0, lbuf1, sbuf1, abuf1, accum, sums,
            sem0, sem1):
    wid = lax.axis_index("c") * NS + lax.axis_index("s")
    batch = wid // TILES_PER_BATCH
    row0 = (wid % TILES_PER_BATCH) * ROWS_PER_TILE
    lbufs, sbufs, abufs = (lbuf0, lbuf1), (sbuf0, sbuf1), (abuf0, abuf1)
    sems = (sem0, sem1)

    zero = jnp.zeros((L,), jnp.float32)
    for k in range(NA):
        accum[pl.ds(k * L, L)] = zero
    lane_iota = jnp.arange(L, dtype=jnp.int32)

    def issue(bi, slot):
        r = row0 + bi * ROWS1
        return (
            pltpu.async_copy(l_hbm.at[batch, 0, pl.ds(r, ROWS1), :],
                             lbufs[slot], sems[slot]),
            pltpu.async_copy(s_hbm.at[batch, 0, pl.ds(r, ROWS1), :],
                             sbufs[slot], sems[slot]),
            pltpu.async_copy(a_hbm.at[batch, pl.ds(r, ROWS1), :],
                             abufs[slot], sems[slot]),
        )

    pending = {0: issue(0, 0)}
    for bi in range(NBLK1):
        slot = bi % 2
        if bi + 1 < NBLK1:
            pending[(bi + 1) % 2] = issue(bi + 1, (bi + 1) % 2)
        for h in pending[slot]:
            h.wait()
        lbuf, sbuf, abuf = lbufs[slot], sbufs[slot], abufs[slot]

        def row_body(r, _):
            def grp_body(g, _):
                for j in range(UNROLL):
                    p = pl.ds((g * UNROLL + j) * L, L)
                    score = (lbuf[r, p] + LAMBDA_L) * (sbuf[r, p] + LAMBDA_S)
                    plsc.addupdate_scatter(
                        accum, [abuf[r, p] * L + lane_iota], score)
                return 0

            lax.fori_loop(0, GPR // UNROLL, grp_body, 0)
            return 0

        lax.fori_loop(0, ROWS1, row_body, 0)

    stride_iota = lane_iota * L
    for k in range(NA // L):
        t = jnp.zeros((L,), jnp.float32)
        for lane in range(L):
            t = t + plsc.load_gather(accum, [stride_iota + (k * L * L + lane)])
        sums[pl.ds(k * L, L)] = t
    pltpu.sync_copy(sums, part_hbm.at[pl.ds(wid * NA, NA)])


@functools.partial(
    pl.kernel,
    mesh=_mesh,
    compiler_params=_params,
    out_type=jax.ShapeDtypeStruct((B, 1, H, W), jnp.float32),
    scratch_types=[
        pltpu.VMEM((ROWS2, W), jnp.float32),
        pltpu.VMEM((ROWS2, W), jnp.float32),
        pltpu.VMEM((ROWS2, W), jnp.int32),
        pltpu.VMEM((ROWS2, W), jnp.float32),
        pltpu.VMEM((ROWS2, W), jnp.float32),
        pltpu.VMEM((ROWS2, W), jnp.int32),
        pltpu.VMEM((ROWS2, W), jnp.float32),
        pltpu.VMEM((ROWS2, W), jnp.float32),
        pltpu.VMEM((TILES_PER_BATCH * NA,), jnp.float32),
        pltpu.VMEM((NA,), jnp.float32),
        pltpu.VMEM((NA,), jnp.float32),
        pltpu.SemaphoreType.DMA,
        pltpu.SemaphoreType.DMA,
        pltpu.SemaphoreType.DMA,
        pltpu.SemaphoreType.DMA,
    ],
)
def _phase2(l_hbm, s_hbm, a_hbm, part_hbm, c_hbm, out_hbm,
            lbuf0, sbuf0, abuf0, lbuf1, sbuf1, abuf1, obuf0, obuf1,
            pbuf, cbuf, fbuf, sem0, sem1, osem0, osem1):
    wid = lax.axis_index("c") * NS + lax.axis_index("s")
    batch = wid // TILES_PER_BATCH
    row0 = (wid % TILES_PER_BATCH) * ROWS_PER_TILE
    lbufs, sbufs, abufs = (lbuf0, lbuf1), (sbuf0, sbuf1), (abuf0, abuf1)
    obufs, sems, osems = (obuf0, obuf1), (sem0, sem1), (osem0, osem1)

    def issue(bi, slot):
        r = row0 + bi * ROWS2
        return (
            pltpu.async_copy(l_hbm.at[batch, 0, pl.ds(r, ROWS2), :],
                             lbufs[slot], sems[slot]),
            pltpu.async_copy(s_hbm.at[batch, 0, pl.ds(r, ROWS2), :],
                             sbufs[slot], sems[slot]),
            pltpu.async_copy(a_hbm.at[batch, pl.ds(r, ROWS2), :],
                             abufs[slot], sems[slot]),
        )

    pending = {0: issue(0, 0)}

    pltpu.sync_copy(part_hbm.at[pl.ds(batch * TILES_PER_BATCH * NA,
                                      TILES_PER_BATCH * NA)], pbuf)
    pltpu.sync_copy(c_hbm, cbuf)
    for k in range(NA // L):
        t = pbuf[pl.ds(k * L, L)]
        for j in range(1, TILES_PER_BATCH):
            t = t + pbuf[pl.ds(j * NA + k * L, L)]
        fbuf[pl.ds(k * L, L)] = cbuf[pl.ds(k * L, L)] / (t + EPS)

    out_pending = {0: None, 1: None}
    for bi in range(NBLK2):
        slot = bi % 2
        if bi + 1 < NBLK2:
            pending[(bi + 1) % 2] = issue(bi + 1, (bi + 1) % 2)
        for h in pending[slot]:
            h.wait()
        if out_pending[slot] is not None:
            out_pending[slot].wait()
        lbuf, sbuf, abuf, obuf = lbufs[slot], sbufs[slot], abufs[slot], obufs[slot]

        def row_body(r, _):
            def grp_body(g, _):
                for j in range(UNROLL):
                    p = pl.ds((g * UNROLL + j) * L, L)
                    score = (lbuf[r, p] + LAMBDA_L) * (sbuf[r, p] + LAMBDA_S)
                    f = plsc.load_gather(fbuf, [abuf[r, p]])
                    obuf[r, p] = score * f
                return 0

            lax.fori_loop(0, GPR // UNROLL, grp_body, 0)
            return 0

        lax.fori_loop(0, ROWS2, row_body, 0)
        r = row0 + bi * ROWS2
        out_pending[slot] = pltpu.async_copy(
            obuf, out_hbm.at[batch, 0, pl.ds(r, ROWS2), :], osems[slot])
    for slot in (0, 1):
        if out_pending[slot] is not None:
            out_pending[slot].wait()


def kernel(lights, settlement, admin_ids, census_totals):
    partials = _phase1(lights, settlement, admin_ids)
    return _phase2(lights, settlement, admin_ids, partials, census_totals)


# speedup vs baseline: 1.8060x; 1.2143x over previous
import functools

import jax
import jax.numpy as jnp
from jax import lax
from jax.experimental import pallas as pl
from jax.experimental.pallas import tpu as pltpu
from jax.experimental.pallas import tpu_sc as plsc

LAMBDA_L = 0.01
LAMBDA_S = 0.01
EPS = 1e-08

B, H, W = 8, 512, 512
NA = 64
NC, NS, L = 2, 16, 16
NW = NC * NS
ROWS_PER_TILE = B * H // NW
TILES_PER_BATCH = NW // B
UNROLL = 4

ROWS1 = 32
NBLK1 = ROWS_PER_TILE // ROWS1
ROWS2 = 16
NBLK2 = ROWS_PER_TILE // ROWS2
GPR = W // L

_mesh = plsc.VectorSubcoreMesh(core_axis_name="c", subcore_axis_name="s")
_params = pltpu.CompilerParams(needs_layout_passes=False)


@functools.partial(
    pl.kernel,
    mesh=_mesh,
    compiler_params=_params,
    out_type=jax.ShapeDtypeStruct((NW * NA,), jnp.float32),
    scratch_types=[
        pltpu.VMEM((ROWS1, W), jnp.float32),
        pltpu.VMEM((ROWS1, W), jnp.float32),
        pltpu.VMEM((ROWS1, W), jnp.int32),
        pltpu.VMEM((ROWS1, W), jnp.float32),
        pltpu.VMEM((ROWS1, W), jnp.float32),
        pltpu.VMEM((ROWS1, W), jnp.int32),
        pltpu.VMEM((L * NA,), jnp.float32),
        pltpu.VMEM((NA,), jnp.float32),
        pltpu.SemaphoreType.DMA,
        pltpu.SemaphoreType.DMA,
    ],
)
def _phase1(l_hbm, s_hbm, a_hbm, part_hbm,
            lbuf0, sbuf0, abuf0, lbuf1, sbuf1, abuf1, accum, sums,
            sem0, sem1):
    wid = lax.axis_index("c") * NS + lax.axis_index("s")
    batch = wid // TILES_PER_BATCH
    row0 = (wid % TILES_PER_BATCH) * ROWS_PER_TILE
    lbufs, sbufs, abufs = (lbuf0, lbuf1), (sbuf0, sbuf1), (abuf0, abuf1)
    sems = (sem0, sem1)

    zero = jnp.zeros((L,), jnp.float32)
    for k in range(NA):
        accum[pl.ds(k * L, L)] = zero
    lane_iota = jnp.arange(L, dtype=jnp.int32)

    def issue(bi, slot):
        r = row0 + bi * ROWS1
        return (
            pltpu.async_copy(l_hbm.at[batch, 0, pl.ds(r, ROWS1), :],
                             lbufs[slot], sems[slot]),
            pltpu.async_copy(s_hbm.at[batch, 0, pl.ds(r, ROWS1), :],
                             sbufs[slot], sems[slot]),
            pltpu.async_copy(a_hbm.at[batch, pl.ds(r, ROWS1), :],
                             abufs[slot], sems[slot]),
        )

    pending = {0: issue(0, 0)}
    for bi in range(NBLK1):
        slot = bi % 2
        if bi + 1 < NBLK1:
            pending[(bi + 1) % 2] = issue(bi + 1, (bi + 1) % 2)
        for h in pending[slot]:
            h.wait()
        lbuf, sbuf, abuf = lbufs[slot], sbufs[slot], abufs[slot]

        def body(i, _):
            off = i * (UNROLL * L)
            r = off // W
            c = off % W
            for j in range(UNROLL):
                p = pl.ds(c + j * L, L)
                score = (lbuf[r, p] + LAMBDA_L) * (sbuf[r, p] + LAMBDA_S)
                plsc.addupdate_scatter(
                    accum, [abuf[r, p] * L + lane_iota], score)
            return 0

        lax.fori_loop(0, ROWS1 * W // (UNROLL * L), body, 0)

    stride_iota = lane_iota * L
    for k in range(NA // L):
        t = jnp.zeros((L,), jnp.float32)
        for lane in range(L):
            t = t + plsc.load_gather(accum, [stride_iota + (k * L * L + lane)])
        sums[pl.ds(k * L, L)] = t
    pltpu.sync_copy(sums, part_hbm.at[pl.ds(wid * NA, NA)])


@functools.partial(
    pl.kernel,
    mesh=_mesh,
    compiler_params=_params,
    out_type=jax.ShapeDtypeStruct((B, 1, H, W), jnp.float32),
    scratch_types=[
        pltpu.VMEM((ROWS2, W), jnp.float32),
        pltpu.VMEM((ROWS2, W), jnp.float32),
        pltpu.VMEM((ROWS2, W), jnp.int32),
        pltpu.VMEM((ROWS2, W), jnp.float32),
        pltpu.VMEM((ROWS2, W), jnp.float32),
        pltpu.VMEM((ROWS2, W), jnp.int32),
        pltpu.VMEM((ROWS2, W), jnp.float32),
        pltpu.VMEM((ROWS2, W), jnp.float32),
        pltpu.VMEM((TILES_PER_BATCH * NA,), jnp.float32),
        pltpu.VMEM((NA,), jnp.float32),
        pltpu.VMEM((NA,), jnp.float32),
        pltpu.SemaphoreType.DMA,
        pltpu.SemaphoreType.DMA,
        pltpu.SemaphoreType.DMA,
        pltpu.SemaphoreType.DMA,
    ],
)
def _phase2(l_hbm, s_hbm, a_hbm, part_hbm, c_hbm, out_hbm,
            lbuf0, sbuf0, abuf0, lbuf1, sbuf1, abuf1, obuf0, obuf1,
            pbuf, cbuf, fbuf, sem0, sem1, osem0, osem1):
    wid = lax.axis_index("c") * NS + lax.axis_index("s")
    batch = wid // TILES_PER_BATCH
    row0 = (wid % TILES_PER_BATCH) * ROWS_PER_TILE
    lbufs, sbufs, abufs = (lbuf0, lbuf1), (sbuf0, sbuf1), (abuf0, abuf1)
    obufs, sems, osems = (obuf0, obuf1), (sem0, sem1), (osem0, osem1)

    def issue(bi, slot):
        r = row0 + bi * ROWS2
        return (
            pltpu.async_copy(l_hbm.at[batch, 0, pl.ds(r, ROWS2), :],
                             lbufs[slot], sems[slot]),
            pltpu.async_copy(s_hbm.at[batch, 0, pl.ds(r, ROWS2), :],
                             sbufs[slot], sems[slot]),
            pltpu.async_copy(a_hbm.at[batch, pl.ds(r, ROWS2), :],
                             abufs[slot], sems[slot]),
        )

    pending = {0: issue(0, 0)}

    pltpu.sync_copy(part_hbm.at[pl.ds(batch * TILES_PER_BATCH * NA,
                                      TILES_PER_BATCH * NA)], pbuf)
    pltpu.sync_copy(c_hbm, cbuf)
    for k in range(NA // L):
        t = pbuf[pl.ds(k * L, L)]
        for j in range(1, TILES_PER_BATCH):
            t = t + pbuf[pl.ds(j * NA + k * L, L)]
        fbuf[pl.ds(k * L, L)] = cbuf[pl.ds(k * L, L)] / (t + EPS)

    out_pending = {0: None, 1: None}
    for bi in range(NBLK2):
        slot = bi % 2
        if bi + 1 < NBLK2:
            pending[(bi + 1) % 2] = issue(bi + 1, (bi + 1) % 2)
        for h in pending[slot]:
            h.wait()
        if out_pending[slot] is not None:
            out_pending[slot].wait()
        lbuf, sbuf, abuf, obuf = lbufs[slot], sbufs[slot], abufs[slot], obufs[slot]

        def body(i, _):
            off = i * (UNROLL * L)
            r = off // W
            c = off % W
            for j in range(UNROLL):
                p = pl.ds(c + j * L, L)
                score = (lbuf[r, p] + LAMBDA_L) * (sbuf[r, p] + LAMBDA_S)
                f = plsc.load_gather(fbuf, [abuf[r, p]])
                obuf[r, p] = score * f
            return 0

        lax.fori_loop(0, ROWS2 * W // (UNROLL * L), body, 0)
        r = row0 + bi * ROWS2
        out_pending[slot] = pltpu.async_copy(
            obuf, out_hbm.at[batch, 0, pl.ds(r, ROWS2), :], osems[slot])
    for slot in (0, 1):
        if out_pending[slot] is not None:
            out_pending[slot].wait()


def kernel(lights, settlement, admin_ids, census_totals):
    partials = _phase1(lights, settlement, admin_ids)
    return _phase2(lights, settlement, admin_ids, partials, census_totals)
